# Initial kernel scaffold; baseline (speedup 1.0000x reference)
import functools

import jax
import jax.numpy as jnp
from jax import lax
from jax.experimental import pallas as pl
from jax.experimental.pallas import tpu as pltpu
from jax.experimental.pallas import tpu_sc as plsc

N = 10000
E = 320000
D_IN = 128
D_HID = 128
N_CLASSES = 64

NUM_SC = 2
NUM_SUBCORES = 16
NUM_TILES = NUM_SC * NUM_SUBCORES

CHUNK = 128
CHUNKS_PER_TILE = 80
E_PAD = NUM_TILES * CHUNKS_PER_TILE * CHUNK
N_PAD = 10240
ROWS_PER_SUBCORE = N_PAD // NUM_SUBCORES

W_AUG = D_HID + 16


def _make_sc_scatter(width):
  mesh = plsc.VectorSubcoreMesh(core_axis_name="c", subcore_axis_name="s")

  @functools.partial(
      pl.kernel,
      out_type=jax.ShapeDtypeStruct((NUM_SC, N_PAD, width), jnp.float32),
      mesh=mesh,
      scratch_types=[
          pltpu.VMEM((CHUNKS_PER_TILE, CHUNK), jnp.int32),
          pltpu.VMEM((CHUNKS_PER_TILE, CHUNK), jnp.int32),
          pltpu.VMEM((CHUNK, width), jnp.float32),
          pltpu.VMEM((CHUNK, width), jnp.float32),
          pltpu.VMEM_SHARED((N_PAD, width), jnp.float32),
          pltpu.SemaphoreType.DMA,
          pltpu.SemaphoreType.DMA,
      ],
  )
  def sc_kernel(t_hbm, src_hbm, dst_hbm, zero_hbm, out_hbm,
                src_v, dst_v, rows0, rows1, acc, sem0, sem1):
    c = lax.axis_index("c")
    s = lax.axis_index("s")
    w = c * NUM_SUBCORES + s

    pltpu.sync_copy(src_hbm.at[w], src_v)
    pltpu.sync_copy(dst_hbm.at[w], dst_v)
    row0 = s * ROWS_PER_SUBCORE
    pltpu.sync_copy(zero_hbm, acc.at[pl.ds(row0, ROWS_PER_SUBCORE)])
    plsc.subcore_barrier()

    pltpu.async_copy(t_hbm.at[src_v.at[0]], rows0, sem0)
    pltpu.async_copy(t_hbm.at[src_v.at[1]], rows1, sem1)

    @pl.loop(0, CHUNKS_PER_TILE - 2, step=2)
    def _(i):
      pltpu.make_async_copy(t_hbm.at[src_v.at[i]], rows0, sem0).wait()
      pltpu.sync_copy(rows0, acc.at[dst_v.at[i]], add=True)
      pltpu.async_copy(t_hbm.at[src_v.at[i + 2]], rows0, sem0)
      pltpu.make_async_copy(t_hbm.at[src_v.at[i + 1]], rows1, sem1).wait()
      pltpu.sync_copy(rows1, acc.at[dst_v.at[i + 1]], add=True)
      pltpu.async_copy(t_hbm.at[src_v.at[i + 3]], rows1, sem1)

    last = CHUNKS_PER_TILE - 2
    pltpu.make_async_copy(t_hbm.at[src_v.at[last]], rows0, sem0).wait()
    pltpu.sync_copy(rows0, acc.at[dst_v.at[last]], add=True)
    pltpu.make_async_copy(t_hbm.at[src_v.at[last + 1]], rows1, sem1).wait()
    pltpu.sync_copy(rows1, acc.at[dst_v.at[last + 1]], add=True)

    plsc.subcore_barrier()
    pltpu.sync_copy(acc.at[pl.ds(row0, ROWS_PER_SUBCORE)],
                    out_hbm.at[c, pl.ds(row0, ROWS_PER_SUBCORE)])

  return sc_kernel


_sc_scatter_aug = _make_sc_scatter(W_AUG)
_sc_scatter_cls = _make_sc_scatter(N_CLASSES)


_BLK = 1024


def _tc_layer1(x_pad, w_neigh, w_self, b):
  def body(x_ref, wn_ref, ws_ref, b_ref, taug_ref, s_ref):
    x = x_ref[...]
    t = jnp.dot(x, wn_ref[...], preferred_element_type=jnp.float32)
    aug = (lax.broadcasted_iota(jnp.int32, (_BLK, 16), 1) == 0).astype(jnp.float32)
    taug_ref[...] = jnp.concatenate([t, aug], axis=1)
    s_ref[...] = jnp.dot(x, ws_ref[...], preferred_element_type=jnp.float32) + b_ref[...]

  return pl.pallas_call(
      body,
      grid=(N_PAD // _BLK,),
      in_specs=[
          pl.BlockSpec((_BLK, D_IN), lambda i: (i, 0)),
          pl.BlockSpec((D_IN, D_HID), lambda i: (0, 0)),
          pl.BlockSpec((D_IN, D_HID), lambda i: (0, 0)),
          pl.BlockSpec((1, D_HID), lambda i: (0, 0)),
      ],
      out_specs=[
          pl.BlockSpec((_BLK, W_AUG), lambda i: (i, 0)),
          pl.BlockSpec((_BLK, D_HID), lambda i: (i, 0)),
      ],
      out_shape=[
          jax.ShapeDtypeStruct((N_PAD, W_AUG), jnp.float32),
          jax.ShapeDtypeStruct((N_PAD, D_HID), jnp.float32),
      ],
  )(x_pad, w_neigh, w_self, b)


def _tc_mid(p0, p1, s1, w_neigh, w_self, b):
  def body(p0_ref, p1_ref, s1_ref, wn_ref, ws_ref, b_ref, t2_ref, s2_ref, r_ref):
    p0v = p0_ref[...]
    p1v = p1_ref[...]
    agg = p0v[:, :D_HID] + p1v[:, :D_HID]
    deg = jnp.sum(p0v[:, D_HID:] + p1v[:, D_HID:], axis=1, keepdims=True)
    recip = 1.0 / jnp.maximum(deg, 1.0)
    h1 = jnp.maximum(s1_ref[...] + agg * recip, 0.0)
    t2_ref[...] = jnp.dot(h1, wn_ref[...], preferred_element_type=jnp.float32)
    s2_ref[...] = (jnp.dot(h1, ws_ref[...], preferred_element_type=jnp.float32)
                   + b_ref[...])
    r_ref[...] = jnp.broadcast_to(recip, (_BLK, N_CLASSES))

  return pl.pallas_call(
      body,
      grid=(N_PAD // _BLK,),
      in_specs=[
          pl.BlockSpec((_BLK, W_AUG), lambda i: (i, 0)),
          pl.BlockSpec((_BLK, W_AUG), lambda i: (i, 0)),
          pl.BlockSpec((_BLK, D_HID), lambda i: (i, 0)),
          pl.BlockSpec((D_HID, N_CLASSES), lambda i: (0, 0)),
          pl.BlockSpec((D_HID, N_CLASSES), lambda i: (0, 0)),
          pl.BlockSpec((1, N_CLASSES), lambda i: (0, 0)),
      ],
      out_specs=[
          pl.BlockSpec((_BLK, N_CLASSES), lambda i: (i, 0)),
          pl.BlockSpec((_BLK, N_CLASSES), lambda i: (i, 0)),
          pl.BlockSpec((_BLK, N_CLASSES), lambda i: (i, 0)),
      ],
      out_shape=[
          jax.ShapeDtypeStruct((N_PAD, N_CLASSES), jnp.float32),
          jax.ShapeDtypeStruct((N_PAD, N_CLASSES), jnp.float32),
          jax.ShapeDtypeStruct((N_PAD, N_CLASSES), jnp.float32),
      ],
  )(p0, p1, s1, w_neigh, w_self, b)


def _tc_final(q0, q1, s2, recip):
  def body(q0_ref, q1_ref, s2_ref, r_ref, o_ref):
    o_ref[...] = s2_ref[...] + (q0_ref[...] + q1_ref[...]) * r_ref[...]

  spec = pl.BlockSpec((_BLK, N_CLASSES), lambda i: (i, 0))
  return pl.pallas_call(
      body,
      grid=(N_PAD // _BLK,),
      in_specs=[spec, spec, spec, spec],
      out_specs=spec,
      out_shape=jax.ShapeDtypeStruct((N_PAD, N_CLASSES), jnp.float32),
  )(q0, q1, s2, recip)


@jax.jit
def kernel(features, edge_index, W_self1, W_neigh1, b1, W_self2, W_neigh2, b2):
  src = edge_index[0].astype(jnp.int32)
  dst = edge_index[1].astype(jnp.int32)
  pad = jnp.full((E_PAD - E,), N, dtype=jnp.int32)
  src_p = jnp.concatenate([src, pad]).reshape(NUM_TILES, CHUNKS_PER_TILE, CHUNK)
  dst_p = jnp.concatenate([dst, pad]).reshape(NUM_TILES, CHUNKS_PER_TILE, CHUNK)

  x_pad = jnp.pad(features, ((0, N_PAD - N), (0, 0)))
  zeros_aug = jnp.zeros((ROWS_PER_SUBCORE, W_AUG), jnp.float32)
  zeros_cls = jnp.zeros((ROWS_PER_SUBCORE, N_CLASSES), jnp.float32)

  t1aug, s1 = _tc_layer1(x_pad, W_neigh1, W_self1, b1.reshape(1, -1))
  parts1 = _sc_scatter_aug(t1aug, src_p, dst_p, zeros_aug)
  t2, s2, recip = _tc_mid(parts1[0], parts1[1], s1,
                          W_neigh2, W_self2, b2.reshape(1, -1))
  parts2 = _sc_scatter_cls(t2, src_p, dst_p, zeros_cls)
  out = _tc_final(parts2[0], parts2[1], s2, recip)
  return out[:N]


# speedup vs baseline: 5.4171x; 5.4171x over previous
import functools

import jax
import jax.numpy as jnp
from jax import lax
from jax.experimental import pallas as pl
from jax.experimental.pallas import tpu as pltpu
from jax.experimental.pallas import tpu_sc as plsc

N = 10000
E = 320000
D_IN = 128
D_HID = 128
N_CLASSES = 64

NUM_SC = 2
NUM_SUBCORES = 16

CHUNK = 128
CHUNKS = 160
E_PAD = NUM_SUBCORES * CHUNKS * CHUNK
N_PAD = 10240
ROWS_PER_SUBCORE = N_PAD // NUM_SUBCORES
DEG_W = 16


def _make_sc_scatter(width, with_deg):
  mesh = plsc.VectorSubcoreMesh(core_axis_name="c", subcore_axis_name="s")

  out_type = [jax.ShapeDtypeStruct((NUM_SC, N_PAD, width), jnp.float32)]
  scratch = [
      pltpu.VMEM((CHUNKS, CHUNK), jnp.int32),
      pltpu.VMEM((CHUNKS, CHUNK), jnp.int32),
      pltpu.VMEM((CHUNK, width), jnp.float32),
      pltpu.VMEM((CHUNK, width), jnp.float32),
      pltpu.VMEM_SHARED((N_PAD, width), jnp.float32),
      pltpu.SemaphoreType.DMA,
      pltpu.SemaphoreType.DMA,
  ]
  if with_deg:
    out_type.append(jax.ShapeDtypeStruct((NUM_SC, N_PAD, DEG_W), jnp.float32))
    scratch += [
        pltpu.VMEM((CHUNK, DEG_W), jnp.float32),
        pltpu.VMEM_SHARED((N_PAD, DEG_W), jnp.float32),
    ]

  def sc_kernel(*refs):
    if with_deg:
      (t_hbm, src_hbm, dst_hbm, zero_hbm, zdeg_hbm, ones_hbm,
       out_hbm, deg_hbm,
       src_v, dst_v, rows0, rows1, acc, sem0, sem1, ones_v, dacc) = refs
    else:
      (t_hbm, src_hbm, dst_hbm, zero_hbm,
       out_hbm,
       src_v, dst_v, rows0, rows1, acc, sem0, sem1) = refs
    c = lax.axis_index("c")
    s = lax.axis_index("s")

    pltpu.sync_copy(src_hbm.at[c, s], src_v)
    pltpu.sync_copy(dst_hbm.at[s], dst_v)
    row0 = s * ROWS_PER_SUBCORE
    pltpu.sync_copy(zero_hbm, acc.at[pl.ds(row0, ROWS_PER_SUBCORE)])
    if with_deg:
      pltpu.sync_copy(zdeg_hbm, dacc.at[pl.ds(row0, ROWS_PER_SUBCORE)])
      pltpu.sync_copy(ones_hbm, ones_v)
    plsc.subcore_barrier()

    def do_deg(i):
      if with_deg:
        @pl.when((i >= CHUNKS // 2) == (c == 1))
        def _():
          pltpu.sync_copy(ones_v, dacc.at[dst_v.at[i]], add=True)

    pltpu.async_copy(t_hbm.at[src_v.at[0]], rows0, sem0)
    pltpu.async_copy(t_hbm.at[src_v.at[1]], rows1, sem1)

    @pl.loop(0, CHUNKS - 2, step=2)
    def _(i):
      pltpu.make_async_copy(t_hbm.at[src_v.at[i]], rows0, sem0).wait()
      pltpu.sync_copy(rows0, acc.at[dst_v.at[i]], add=True)
      do_deg(i)
      pltpu.async_copy(t_hbm.at[src_v.at[i + 2]], rows0, sem0)
      pltpu.make_async_copy(t_hbm.at[src_v.at[i + 1]], rows1, sem1).wait()
      pltpu.sync_copy(rows1, acc.at[dst_v.at[i + 1]], add=True)
      do_deg(i + 1)
      pltpu.async_copy(t_hbm.at[src_v.at[i + 3]], rows1, sem1)

    last = CHUNKS - 2
    pltpu.make_async_copy(t_hbm.at[src_v.at[last]], rows0, sem0).wait()
    pltpu.sync_copy(rows0, acc.at[dst_v.at[last]], add=True)
    do_deg(last)
    pltpu.make_async_copy(t_hbm.at[src_v.at[last + 1]], rows1, sem1).wait()
    pltpu.sync_copy(rows1, acc.at[dst_v.at[last + 1]], add=True)
    do_deg(last + 1)

    plsc.subcore_barrier()
    pltpu.sync_copy(acc.at[pl.ds(row0, ROWS_PER_SUBCORE)],
                    out_hbm.at[c, pl.ds(row0, ROWS_PER_SUBCORE)])
    if with_deg:
      pltpu.sync_copy(dacc.at[pl.ds(row0, ROWS_PER_SUBCORE)],
                      deg_hbm.at[c, pl.ds(row0, ROWS_PER_SUBCORE)])

  return pl.kernel(
      sc_kernel,
      out_type=out_type,
      mesh=mesh,
      compiler_params=pltpu.CompilerParams(use_tc_tiling_on_sc=False),
      scratch_types=scratch,
  )


_sc_scatter_l1 = _make_sc_scatter(D_HID // 2, with_deg=True)
_sc_scatter_l2 = _make_sc_scatter(N_CLASSES // 2, with_deg=False)


_BLK = 1024
_HALF1 = D_HID // 2
_HALF2 = N_CLASSES // 2


def _tc_layer1(x_pad, w_neigh, w_self, b):
  def body(x_ref, wn_ref, ws_ref, b_ref, ta_ref, tb_ref, s_ref):
    x = x_ref[...]
    wn = wn_ref[...]
    ta_ref[...] = jnp.dot(x, wn[:, :_HALF1], preferred_element_type=jnp.float32)
    tb_ref[...] = jnp.dot(x, wn[:, _HALF1:], preferred_element_type=jnp.float32)
    s_ref[...] = jnp.dot(x, ws_ref[...], preferred_element_type=jnp.float32) + b_ref[...]

  return pl.pallas_call(
      body,
      grid=(N_PAD // _BLK,),
      in_specs=[
          pl.BlockSpec((_BLK, D_IN), lambda i: (i, 0)),
          pl.BlockSpec((D_IN, D_HID), lambda i: (0, 0)),
          pl.BlockSpec((D_IN, D_HID), lambda i: (0, 0)),
          pl.BlockSpec((1, D_HID), lambda i: (0, 0)),
      ],
      out_specs=[
          pl.BlockSpec((_BLK, _HALF1), lambda i: (i, 0)),
          pl.BlockSpec((_BLK, _HALF1), lambda i: (i, 0)),
          pl.BlockSpec((_BLK, D_HID), lambda i: (i, 0)),
      ],
      out_shape=[
          jax.ShapeDtypeStruct((N_PAD, _HALF1), jnp.float32),
          jax.ShapeDtypeStruct((N_PAD, _HALF1), jnp.float32),
          jax.ShapeDtypeStruct((N_PAD, D_HID), jnp.float32),
      ],
  )(x_pad, w_neigh, w_self, b)


def _tc_mid(pa, pb, d0, d1, s1, w_neigh, w_self, b):
  def body(pa_ref, pb_ref, d0_ref, d1_ref, s1_ref, wn_ref, ws_ref, b_ref,
           ta_ref, tb_ref, s2_ref, r_ref):
    agg = jnp.concatenate([pa_ref[...], pb_ref[...]], axis=1)
    deg = (d0_ref[...] + d1_ref[...])[:, :1]
    recip = 1.0 / jnp.maximum(deg, 1.0)
    h1 = jnp.maximum(s1_ref[...] + agg * recip, 0.0)
    wn = wn_ref[...]
    ta_ref[...] = jnp.dot(h1, wn[:, :_HALF2], preferred_element_type=jnp.float32)
    tb_ref[...] = jnp.dot(h1, wn[:, _HALF2:], preferred_element_type=jnp.float32)
    s2_ref[...] = (jnp.dot(h1, ws_ref[...], preferred_element_type=jnp.float32)
                   + b_ref[...])
    r_ref[...] = jnp.broadcast_to(recip, (_BLK, N_CLASSES))

  return pl.pallas_call(
      body,
      grid=(N_PAD // _BLK,),
      in_specs=[
          pl.BlockSpec((_BLK, _HALF1), lambda i: (i, 0)),
          pl.BlockSpec((_BLK, _HALF1), lambda i: (i, 0)),
          pl.BlockSpec((_BLK, DEG_W), lambda i: (i, 0)),
          pl.BlockSpec((_BLK, DEG_W), lambda i: (i, 0)),
          pl.BlockSpec((_BLK, D_HID), lambda i: (i, 0)),
          pl.BlockSpec((D_HID, N_CLASSES), lambda i: (0, 0)),
          pl.BlockSpec((D_HID, N_CLASSES), lambda i: (0, 0)),
          pl.BlockSpec((1, N_CLASSES), lambda i: (0, 0)),
      ],
      out_specs=[
          pl.BlockSpec((_BLK, _HALF2), lambda i: (i, 0)),
          pl.BlockSpec((_BLK, _HALF2), lambda i: (i, 0)),
          pl.BlockSpec((_BLK, N_CLASSES), lambda i: (i, 0)),
          pl.BlockSpec((_BLK, N_CLASSES), lambda i: (i, 0)),
      ],
      out_shape=[
          jax.ShapeDtypeStruct((N_PAD, _HALF2), jnp.float32),
          jax.ShapeDtypeStruct((N_PAD, _HALF2), jnp.float32),
          jax.ShapeDtypeStruct((N_PAD, N_CLASSES), jnp.float32),
          jax.ShapeDtypeStruct((N_PAD, N_CLASSES), jnp.float32),
      ],
  )(pa, pb, d0, d1, s1, w_neigh, w_self, b)


def _tc_final(qa, qb, s2, recip):
  def body(qa_ref, qb_ref, s2_ref, r_ref, o_ref):
    agg = jnp.concatenate([qa_ref[...], qb_ref[...]], axis=1)
    o_ref[...] = s2_ref[...] + agg * r_ref[...]

  half = pl.BlockSpec((_BLK, _HALF2), lambda i: (i, 0))
  full = pl.BlockSpec((_BLK, N_CLASSES), lambda i: (i, 0))
  return pl.pallas_call(
      body,
      grid=(N_PAD // _BLK,),
      in_specs=[half, half, full, full],
      out_specs=full,
      out_shape=jax.ShapeDtypeStruct((N_PAD, N_CLASSES), jnp.float32),
  )(qa, qb, s2, recip)


@jax.jit
def kernel(features, edge_index, W_self1, W_neigh1, b1, W_self2, W_neigh2, b2):
  src = edge_index[0].astype(jnp.int32)
  dst = edge_index[1].astype(jnp.int32)
  pad = jnp.full((E_PAD - E,), N, dtype=jnp.int32)
  src_p = jnp.concatenate([src, pad]).reshape(NUM_SUBCORES, CHUNKS, CHUNK)
  src_stack = jnp.stack([src_p, src_p + N_PAD])
  dst_p = jnp.concatenate([dst, pad]).reshape(NUM_SUBCORES, CHUNKS, CHUNK)

  x_pad = jnp.pad(features, ((0, N_PAD - N), (0, 0)))
  zeros1 = jnp.zeros((ROWS_PER_SUBCORE, _HALF1), jnp.float32)
  zeros2 = jnp.zeros((ROWS_PER_SUBCORE, _HALF2), jnp.float32)
  zerosd = jnp.zeros((ROWS_PER_SUBCORE, DEG_W), jnp.float32)
  ones = jnp.ones((CHUNK, DEG_W), jnp.float32)

  t1a, t1b, s1 = _tc_layer1(x_pad, W_neigh1, W_self1, b1.reshape(1, -1))
  t1cat = jnp.concatenate([t1a, t1b], axis=0)
  p1, degp = _sc_scatter_l1(t1cat, src_stack, dst_p, zeros1, zerosd, ones)
  t2a, t2b, s2, recip = _tc_mid(p1[0], p1[1], degp[0], degp[1], s1,
                                W_neigh2, W_self2, b2.reshape(1, -1))
  t2cat = jnp.concatenate([t2a, t2b], axis=0)
  (p2,) = _sc_scatter_l2(t2cat, src_stack, dst_p, zeros2)
  out = _tc_final(p2[0], p2[1], s2, recip)
  return out[:N]


# speedup vs baseline: 9.5538x; 1.7636x over previous
import functools

import jax
import jax.numpy as jnp
from jax import lax
from jax.experimental import pallas as pl
from jax.experimental.pallas import tpu as pltpu
from jax.experimental.pallas import tpu_sc as plsc

N = 10000
E = 320000
D_IN = 128
D_HID = 128
N_CLASSES = 64

NUM_SC = 2
NUM_SUBCORES = 16

CHUNK = 128
CHUNKS = 160
E_PAD = NUM_SUBCORES * CHUNKS * CHUNK
N_PAD = 10240
ROWS_PER_SUBCORE = N_PAD // NUM_SUBCORES
DEG_W = 16


def _make_sc_scatter(width, with_deg):
  mesh = plsc.VectorSubcoreMesh(core_axis_name="c", subcore_axis_name="s")

  out_type = [jax.ShapeDtypeStruct((NUM_SC, N_PAD, width), jnp.float32)]
  scratch = [
      pltpu.VMEM((CHUNKS, CHUNK), jnp.int32),
      pltpu.VMEM((CHUNKS, CHUNK), jnp.int32),
      pltpu.VMEM((CHUNK, width), jnp.float32),
      pltpu.VMEM((CHUNK, width), jnp.float32),
      pltpu.VMEM_SHARED((N_PAD, width), jnp.float32),
      pltpu.SemaphoreType.DMA,
      pltpu.SemaphoreType.DMA,
  ]
  if with_deg:
    out_type.append(jax.ShapeDtypeStruct((NUM_SC, N_PAD, DEG_W), jnp.float32))
    scratch += [
        pltpu.VMEM((CHUNK, DEG_W), jnp.float32),
        pltpu.VMEM_SHARED((N_PAD, DEG_W), jnp.float32),
    ]

  def sc_kernel(*refs):
    if with_deg:
      (t_hbm, src_hbm, dst_hbm, zero_hbm, zdeg_hbm, ones_hbm,
       out_hbm, deg_hbm,
       src_v, dst_v, rows0, rows1, acc, sem0, sem1, ones_v, dacc) = refs
    else:
      (t_hbm, src_hbm, dst_hbm, zero_hbm,
       out_hbm,
       src_v, dst_v, rows0, rows1, acc, sem0, sem1) = refs
    c = lax.axis_index("c")
    s = lax.axis_index("s")

    pltpu.sync_copy(src_hbm.at[c, s], src_v)
    pltpu.sync_copy(dst_hbm.at[s], dst_v)
    row0 = s * ROWS_PER_SUBCORE
    pltpu.sync_copy(zero_hbm, acc.at[pl.ds(row0, ROWS_PER_SUBCORE)])
    if with_deg:
      pltpu.sync_copy(zdeg_hbm, dacc.at[pl.ds(row0, ROWS_PER_SUBCORE)])
      pltpu.sync_copy(ones_hbm, ones_v)
    plsc.subcore_barrier()

    def do_deg(i):
      if with_deg:
        @pl.when((i >= CHUNKS // 2) == (c == 1))
        def _():
          pltpu.sync_copy(ones_v, dacc.at[dst_v.at[i]], add=True)

    pltpu.async_copy(t_hbm.at[src_v.at[0]], rows0, sem0)
    pltpu.async_copy(t_hbm.at[src_v.at[1]], rows1, sem1)

    @pl.loop(0, CHUNKS - 2, step=2)
    def _(i):
      pltpu.make_async_copy(t_hbm.at[src_v.at[i]], rows0, sem0).wait()
      pltpu.sync_copy(rows0, acc.at[dst_v.at[i]], add=True)
      do_deg(i)
      pltpu.async_copy(t_hbm.at[src_v.at[i + 2]], rows0, sem0)
      pltpu.make_async_copy(t_hbm.at[src_v.at[i + 1]], rows1, sem1).wait()
      pltpu.sync_copy(rows1, acc.at[dst_v.at[i + 1]], add=True)
      do_deg(i + 1)
      pltpu.async_copy(t_hbm.at[src_v.at[i + 3]], rows1, sem1)

    last = CHUNKS - 2
    pltpu.make_async_copy(t_hbm.at[src_v.at[last]], rows0, sem0).wait()
    pltpu.sync_copy(rows0, acc.at[dst_v.at[last]], add=True)
    do_deg(last)
    pltpu.make_async_copy(t_hbm.at[src_v.at[last + 1]], rows1, sem1).wait()
    pltpu.sync_copy(rows1, acc.at[dst_v.at[last + 1]], add=True)
    do_deg(last + 1)

    plsc.subcore_barrier()
    pltpu.sync_copy(acc.at[pl.ds(row0, ROWS_PER_SUBCORE)],
                    out_hbm.at[c, pl.ds(row0, ROWS_PER_SUBCORE)])
    if with_deg:
      pltpu.sync_copy(dacc.at[pl.ds(row0, ROWS_PER_SUBCORE)],
                      deg_hbm.at[c, pl.ds(row0, ROWS_PER_SUBCORE)])

  return pl.kernel(
      sc_kernel,
      out_type=out_type,
      mesh=mesh,
      compiler_params=pltpu.CompilerParams(use_tc_tiling_on_sc=False),
      scratch_types=scratch,
  )


_sc_scatter_l1 = _make_sc_scatter(D_HID // 2, with_deg=True)
_sc_scatter_l2 = _make_sc_scatter(N_CLASSES // 2, with_deg=False)


_BLK = 1024
_HALF1 = D_HID // 2
_HALF2 = N_CLASSES // 2


def _tc_layer1(x_pad, w_neigh, w_self, b):
  def body(x_ref, wn_ref, ws_ref, b_ref, ta_ref, tb_ref, s_ref):
    x = x_ref[...]
    wn = wn_ref[...]
    ta_ref[...] = jnp.dot(x, wn[:, :_HALF1], preferred_element_type=jnp.float32)
    tb_ref[...] = jnp.dot(x, wn[:, _HALF1:], preferred_element_type=jnp.float32)
    s_ref[...] = jnp.dot(x, ws_ref[...], preferred_element_type=jnp.float32) + b_ref[...]

  return pl.pallas_call(
      body,
      grid=(N_PAD // _BLK,),
      in_specs=[
          pl.BlockSpec((_BLK, D_IN), lambda i: (i, 0)),
          pl.BlockSpec((D_IN, D_HID), lambda i: (0, 0)),
          pl.BlockSpec((D_IN, D_HID), lambda i: (0, 0)),
          pl.BlockSpec((1, D_HID), lambda i: (0, 0)),
      ],
      out_specs=[
          pl.BlockSpec((_BLK, _HALF1), lambda i: (i, 0)),
          pl.BlockSpec((_BLK, _HALF1), lambda i: (i, 0)),
          pl.BlockSpec((_BLK, D_HID), lambda i: (i, 0)),
      ],
      out_shape=[
          jax.ShapeDtypeStruct((N_PAD, _HALF1), jnp.float32),
          jax.ShapeDtypeStruct((N_PAD, _HALF1), jnp.float32),
          jax.ShapeDtypeStruct((N_PAD, D_HID), jnp.float32),
      ],
  )(x_pad, w_neigh, w_self, b)


def _tc_mid(pa, pb, d0, d1, s1, w_neigh, w_self, b):
  def body(pa_ref, pb_ref, d0_ref, d1_ref, s1_ref, wn_ref, ws_ref, b_ref,
           ta_ref, tb_ref, s2_ref, r_ref):
    agg = jnp.concatenate([pa_ref[...], pb_ref[...]], axis=1)
    deg = (d0_ref[...] + d1_ref[...])[:, :1]
    recip = 1.0 / jnp.maximum(deg, 1.0)
    h1 = jnp.maximum(s1_ref[...] + agg * recip, 0.0)
    wn = wn_ref[...]
    ta_ref[...] = jnp.dot(h1, wn[:, :_HALF2], preferred_element_type=jnp.float32)
    tb_ref[...] = jnp.dot(h1, wn[:, _HALF2:], preferred_element_type=jnp.float32)
    s2_ref[...] = (jnp.dot(h1, ws_ref[...], preferred_element_type=jnp.float32)
                   + b_ref[...])
    r_ref[...] = jnp.broadcast_to(recip, (_BLK, N_CLASSES))

  return pl.pallas_call(
      body,
      grid=(N_PAD // _BLK,),
      in_specs=[
          pl.BlockSpec((_BLK, _HALF1), lambda i: (i, 0)),
          pl.BlockSpec((_BLK, _HALF1), lambda i: (i, 0)),
          pl.BlockSpec((_BLK, DEG_W), lambda i: (i, 0)),
          pl.BlockSpec((_BLK, DEG_W), lambda i: (i, 0)),
          pl.BlockSpec((_BLK, D_HID), lambda i: (i, 0)),
          pl.BlockSpec((D_HID, N_CLASSES), lambda i: (0, 0)),
          pl.BlockSpec((D_HID, N_CLASSES), lambda i: (0, 0)),
          pl.BlockSpec((1, N_CLASSES), lambda i: (0, 0)),
      ],
      out_specs=[
          pl.BlockSpec((_BLK, _HALF2), lambda i: (i, 0)),
          pl.BlockSpec((_BLK, _HALF2), lambda i: (i, 0)),
          pl.BlockSpec((_BLK, N_CLASSES), lambda i: (i, 0)),
          pl.BlockSpec((_BLK, N_CLASSES), lambda i: (i, 0)),
      ],
      out_shape=[
          jax.ShapeDtypeStruct((N_PAD, _HALF2), jnp.float32),
          jax.ShapeDtypeStruct((N_PAD, _HALF2), jnp.float32),
          jax.ShapeDtypeStruct((N_PAD, N_CLASSES), jnp.float32),
          jax.ShapeDtypeStruct((N_PAD, N_CLASSES), jnp.float32),
      ],
  )(pa, pb, d0, d1, s1, w_neigh, w_self, b)


def _tc_final(qa, qb, s2, recip):
  def body(qa_ref, qb_ref, s2_ref, r_ref, o_ref):
    agg = jnp.concatenate([qa_ref[...], qb_ref[...]], axis=1)
    o_ref[...] = s2_ref[...] + agg * r_ref[...]

  half = pl.BlockSpec((_BLK, _HALF2), lambda i: (i, 0))
  full = pl.BlockSpec((_BLK, N_CLASSES), lambda i: (i, 0))
  return pl.pallas_call(
      body,
      grid=(N_PAD // _BLK,),
      in_specs=[half, half, full, full],
      out_specs=full,
      out_shape=jax.ShapeDtypeStruct((N_PAD, N_CLASSES), jnp.float32),
  )(qa, qb, s2, recip)


@jax.jit
def kernel(features, edge_index, W_self1, W_neigh1, b1, W_self2, W_neigh2, b2):
  src = edge_index[0].astype(jnp.int32)
  dst = edge_index[1].astype(jnp.int32)
  pad = N + jax.lax.iota(jnp.int32, E_PAD - E) % (N_PAD - N)
  src_p = jnp.concatenate([src, pad]).reshape(NUM_SUBCORES, CHUNKS, CHUNK)
  src_stack = jnp.stack([src_p, src_p + N_PAD])
  dst_p = jnp.concatenate([dst, pad]).reshape(NUM_SUBCORES, CHUNKS, CHUNK)

  x_pad = jnp.pad(features, ((0, N_PAD - N), (0, 0)))
  zeros1 = jnp.zeros((ROWS_PER_SUBCORE, _HALF1), jnp.float32)
  zeros2 = jnp.zeros((ROWS_PER_SUBCORE, _HALF2), jnp.float32)
  zerosd = jnp.zeros((ROWS_PER_SUBCORE, DEG_W), jnp.float32)
  ones = jnp.ones((CHUNK, DEG_W), jnp.float32)

  t1a, t1b, s1 = _tc_layer1(x_pad, W_neigh1, W_self1, b1.reshape(1, -1))
  t1cat = jnp.concatenate([t1a, t1b], axis=0)
  p1, degp = _sc_scatter_l1(t1cat, src_stack, dst_p, zeros1, zerosd, ones)
  t2a, t2b, s2, recip = _tc_mid(p1[0], p1[1], degp[0], degp[1], s1,
                                W_neigh2, W_self2, b2.reshape(1, -1))
  t2cat = jnp.concatenate([t2a, t2b], axis=0)
  (p2,) = _sc_scatter_l2(t2cat, src_stack, dst_p, zeros2)
  out = _tc_final(p2[0], p2[1], s2, recip)
  return out[:N]


# speedup vs baseline: 10.6214x; 1.1117x over previous
import functools

import jax
import jax.numpy as jnp
from jax import lax
from jax.experimental import pallas as pl
from jax.experimental.pallas import tpu as pltpu
from jax.experimental.pallas import tpu_sc as plsc

N = 10000
E = 320000
D_IN = 128
D_HID = 128
N_CLASSES = 64

NUM_SC = 2
NUM_SUBCORES = 16
NUM_TILES = NUM_SC * NUM_SUBCORES

CHUNK = 64
CHUNKS = 160
E_PAD = NUM_TILES * CHUNKS * CHUNK
N_PAD = 10240
ROWS_PER_SUBCORE = N_PAD // NUM_SUBCORES
DEG_W = 16


def _make_sc_scatter(width, with_deg):
  mesh = plsc.VectorSubcoreMesh(core_axis_name="c", subcore_axis_name="s")

  out_type = [jax.ShapeDtypeStruct((NUM_SC, N_PAD, width), jnp.float32)]
  scratch = [
      pltpu.VMEM((CHUNKS, CHUNK), jnp.int32),
      pltpu.VMEM((CHUNKS, CHUNK), jnp.int32),
      pltpu.VMEM((CHUNK, width), jnp.float32),
      pltpu.VMEM((CHUNK, width), jnp.float32),
      pltpu.VMEM_SHARED((N_PAD, width), jnp.float32),
      pltpu.SemaphoreType.DMA,
      pltpu.SemaphoreType.DMA,
  ]
  if with_deg:
    out_type.append(jax.ShapeDtypeStruct((NUM_SC, N_PAD, DEG_W), jnp.float32))
    scratch += [
        pltpu.VMEM((CHUNK, DEG_W), jnp.float32),
        pltpu.VMEM_SHARED((N_PAD, DEG_W), jnp.float32),
    ]

  def sc_kernel(*refs):
    if with_deg:
      (t_hbm, src_hbm, dst_hbm, zero_hbm, zdeg_hbm, ones_hbm,
       out_hbm, deg_hbm,
       src_v, dst_v, rows0, rows1, acc, sem0, sem1, ones_v, dacc) = refs
    else:
      (t_hbm, src_hbm, dst_hbm, zero_hbm,
       out_hbm,
       src_v, dst_v, rows0, rows1, acc, sem0, sem1) = refs
    c = lax.axis_index("c")
    s = lax.axis_index("s")
    w = c * NUM_SUBCORES + s

    pltpu.sync_copy(src_hbm.at[w], src_v)
    pltpu.sync_copy(dst_hbm.at[w], dst_v)
    row0 = s * ROWS_PER_SUBCORE
    pltpu.sync_copy(zero_hbm, acc.at[pl.ds(row0, ROWS_PER_SUBCORE)])
    if with_deg:
      pltpu.sync_copy(zdeg_hbm, dacc.at[pl.ds(row0, ROWS_PER_SUBCORE)])
      pltpu.sync_copy(ones_hbm, ones_v)
    plsc.subcore_barrier()

    def do_deg(i):
      if with_deg:
        pltpu.sync_copy(ones_v, dacc.at[dst_v.at[i]], add=True)

    pltpu.async_copy(t_hbm.at[src_v.at[0]], rows0, sem0)
    pltpu.async_copy(t_hbm.at[src_v.at[1]], rows1, sem1)

    @pl.loop(0, CHUNKS - 2, step=2)
    def _(i):
      pltpu.make_async_copy(t_hbm.at[src_v.at[i]], rows0, sem0).wait()
      pltpu.sync_copy(rows0, acc.at[dst_v.at[i]], add=True)
      do_deg(i)
      pltpu.async_copy(t_hbm.at[src_v.at[i + 2]], rows0, sem0)
      pltpu.make_async_copy(t_hbm.at[src_v.at[i + 1]], rows1, sem1).wait()
      pltpu.sync_copy(rows1, acc.at[dst_v.at[i + 1]], add=True)
      do_deg(i + 1)
      pltpu.async_copy(t_hbm.at[src_v.at[i + 3]], rows1, sem1)

    last = CHUNKS - 2
    pltpu.make_async_copy(t_hbm.at[src_v.at[last]], rows0, sem0).wait()
    pltpu.sync_copy(rows0, acc.at[dst_v.at[last]], add=True)
    do_deg(last)
    pltpu.make_async_copy(t_hbm.at[src_v.at[last + 1]], rows1, sem1).wait()
    pltpu.sync_copy(rows1, acc.at[dst_v.at[last + 1]], add=True)
    do_deg(last + 1)

    plsc.subcore_barrier()
    pltpu.sync_copy(acc.at[pl.ds(row0, ROWS_PER_SUBCORE)],
                    out_hbm.at[c, pl.ds(row0, ROWS_PER_SUBCORE)])
    if with_deg:
      pltpu.sync_copy(dacc.at[pl.ds(row0, ROWS_PER_SUBCORE)],
                      deg_hbm.at[c, pl.ds(row0, ROWS_PER_SUBCORE)])

  return pl.kernel(
      sc_kernel,
      out_type=out_type,
      mesh=mesh,
      compiler_params=pltpu.CompilerParams(use_tc_tiling_on_sc=False),
      scratch_types=scratch,
  )


_sc_scatter_l1 = _make_sc_scatter(D_HID, with_deg=True)
_sc_scatter_l2 = _make_sc_scatter(N_CLASSES, with_deg=False)


_BLK = 1024


def _tc_layer1(x_pad, w_neigh, w_self, b):
  def body(x_ref, wn_ref, ws_ref, b_ref, t_ref, s_ref):
    x = x_ref[...]
    t_ref[...] = jnp.dot(x, wn_ref[...], preferred_element_type=jnp.float32)
    s_ref[...] = jnp.dot(x, ws_ref[...], preferred_element_type=jnp.float32) + b_ref[...]

  return pl.pallas_call(
      body,
      grid=(N_PAD // _BLK,),
      in_specs=[
          pl.BlockSpec((_BLK, D_IN), lambda i: (i, 0)),
          pl.BlockSpec((D_IN, D_HID), lambda i: (0, 0)),
          pl.BlockSpec((D_IN, D_HID), lambda i: (0, 0)),
          pl.BlockSpec((1, D_HID), lambda i: (0, 0)),
      ],
      out_specs=[
          pl.BlockSpec((_BLK, D_HID), lambda i: (i, 0)),
          pl.BlockSpec((_BLK, D_HID), lambda i: (i, 0)),
      ],
      out_shape=[
          jax.ShapeDtypeStruct((N_PAD, D_HID), jnp.float32),
          jax.ShapeDtypeStruct((N_PAD, D_HID), jnp.float32),
      ],
  )(x_pad, w_neigh, w_self, b)


def _tc_mid(p0, p1, d0, d1, s1, w_neigh, w_self, b):
  def body(p0_ref, p1_ref, d0_ref, d1_ref, s1_ref, wn_ref, ws_ref, b_ref,
           t2_ref, s2_ref, r_ref):
    agg = p0_ref[...] + p1_ref[...]
    deg = (d0_ref[...] + d1_ref[...])[:, :1]
    recip = 1.0 / jnp.maximum(deg, 1.0)
    h1 = jnp.maximum(s1_ref[...] + agg * recip, 0.0)
    t2_ref[...] = jnp.dot(h1, wn_ref[...], preferred_element_type=jnp.float32)
    s2_ref[...] = (jnp.dot(h1, ws_ref[...], preferred_element_type=jnp.float32)
                   + b_ref[...])
    r_ref[...] = jnp.broadcast_to(recip, (_BLK, N_CLASSES))

  return pl.pallas_call(
      body,
      grid=(N_PAD // _BLK,),
      in_specs=[
          pl.BlockSpec((_BLK, D_HID), lambda i: (i, 0)),
          pl.BlockSpec((_BLK, D_HID), lambda i: (i, 0)),
          pl.BlockSpec((_BLK, DEG_W), lambda i: (i, 0)),
          pl.BlockSpec((_BLK, DEG_W), lambda i: (i, 0)),
          pl.BlockSpec((_BLK, D_HID), lambda i: (i, 0)),
          pl.BlockSpec((D_HID, N_CLASSES), lambda i: (0, 0)),
          pl.BlockSpec((D_HID, N_CLASSES), lambda i: (0, 0)),
          pl.BlockSpec((1, N_CLASSES), lambda i: (0, 0)),
      ],
      out_specs=[
          pl.BlockSpec((_BLK, N_CLASSES), lambda i: (i, 0)),
          pl.BlockSpec((_BLK, N_CLASSES), lambda i: (i, 0)),
          pl.BlockSpec((_BLK, N_CLASSES), lambda i: (i, 0)),
      ],
      out_shape=[
          jax.ShapeDtypeStruct((N_PAD, N_CLASSES), jnp.float32),
          jax.ShapeDtypeStruct((N_PAD, N_CLASSES), jnp.float32),
          jax.ShapeDtypeStruct((N_PAD, N_CLASSES), jnp.float32),
      ],
  )(p0, p1, d0, d1, s1, w_neigh, w_self, b)


def _tc_final(q0, q1, s2, recip):
  def body(q0_ref, q1_ref, s2_ref, r_ref, o_ref):
    o_ref[...] = s2_ref[...] + (q0_ref[...] + q1_ref[...]) * r_ref[...]

  spec = pl.BlockSpec((_BLK, N_CLASSES), lambda i: (i, 0))
  return pl.pallas_call(
      body,
      grid=(N_PAD // _BLK,),
      in_specs=[spec, spec, spec, spec],
      out_specs=spec,
      out_shape=jax.ShapeDtypeStruct((N_PAD, N_CLASSES), jnp.float32),
  )(q0, q1, s2, recip)


@jax.jit
def kernel(features, edge_index, W_self1, W_neigh1, b1, W_self2, W_neigh2, b2):
  src = edge_index[0].astype(jnp.int32)
  dst = edge_index[1].astype(jnp.int32)
  pad = N + jax.lax.iota(jnp.int32, E_PAD - E) % (N_PAD - N)
  src_p = jnp.concatenate([src, pad]).reshape(NUM_TILES, CHUNKS, CHUNK)
  dst_p = jnp.concatenate([dst, pad]).reshape(NUM_TILES, CHUNKS, CHUNK)

  x_pad = jnp.pad(features, ((0, N_PAD - N), (0, 0)))
  zeros1 = jnp.zeros((ROWS_PER_SUBCORE, D_HID), jnp.float32)
  zeros2 = jnp.zeros((ROWS_PER_SUBCORE, N_CLASSES), jnp.float32)
  zerosd = jnp.zeros((ROWS_PER_SUBCORE, DEG_W), jnp.float32)
  ones = jnp.ones((CHUNK, DEG_W), jnp.float32)

  t1, s1 = _tc_layer1(x_pad, W_neigh1, W_self1, b1.reshape(1, -1))
  p1, degp = _sc_scatter_l1(t1, src_p, dst_p, zeros1, zerosd, ones)
  t2, s2, recip = _tc_mid(p1[0], p1[1], degp[0], degp[1], s1,
                          W_neigh2, W_self2, b2.reshape(1, -1))
  (p2,) = _sc_scatter_l2(t2, src_p, dst_p, zeros2)
  out = _tc_final(p2[0], p2[1], s2, recip)
  return out[:N]
